# baseline (speedup 1.0000x reference)
import jax
import jax.numpy as jnp
from jax import lax
from jax.experimental import pallas as pl
from jax.experimental.pallas import tpu as pltpu
from jax.experimental.pallas import tpu_sc as plsc

B = 32
S = 256
T = 1024
MASK = 768
D = 768
ROWS = B * S
CHUNK = 32
NCHUNK = T // CHUNK
DEPTH = 4
LANES = 16
NC = 2
NS = 16


def _body(table, idx, out, idx_v, src_v,
          rows_0, rows_1, rows_2, rows_3,
          gsem_0, gsem_1, gsem_2, gsem_3,
          wsem_0, wsem_1, wsem_2, wsem_3):
    wid = lax.axis_index("s") * NC + lax.axis_index("c")
    pltpu.sync_copy(idx, idx_v)

    in_base = wid * S - MASK
    for i in range(T // LANES):
        v = idx_v[pl.ds(i * LANES, LANES)]
        src = jnp.where(v < MASK, jnp.int32(ROWS), v + in_base)
        src_v[i * LANES // CHUNK, pl.ds((i * LANES) % CHUNK, LANES)] = src

    bufs = (rows_0, rows_1, rows_2, rows_3)
    gsems = (gsem_0, gsem_1, gsem_2, gsem_3)
    wsems = (wsem_0, wsem_1, wsem_2, wsem_3)
    gathers = [None] * NCHUNK
    writes = [None] * NCHUNK

    def start_gather(c):
        gathers[c] = pltpu.async_copy(
            table.at[src_v.at[c]], bufs[c % DEPTH], gsems[c % DEPTH])

    for c in range(min(DEPTH, NCHUNK)):
        start_gather(c)
    for c in range(NCHUNK):
        gathers[c].wait()
        writes[c] = pltpu.async_copy(
            bufs[c % DEPTH], out.at[pl.ds(wid * T + c * CHUNK, CHUNK)],
            wsems[c % DEPTH])
        nxt = c + DEPTH
        if nxt < NCHUNK:
            writes[c].wait()
            start_gather(nxt)
        else:
            writes[c].wait()


def kernel(inputs, mask_indices, un_masked_indices, mst):
    idx = jnp.concatenate([mask_indices, un_masked_indices]).astype(jnp.int32)
    table = jnp.concatenate(
        [inputs.reshape(ROWS, D),
         jnp.broadcast_to(mst.reshape(1, D).astype(inputs.dtype), (8, D))],
        axis=0)
    mesh = plsc.VectorSubcoreMesh(core_axis_name="c", subcore_axis_name="s")
    out = pl.kernel(
        _body,
        mesh=mesh,
        out_type=jax.ShapeDtypeStruct((B * T, D), inputs.dtype),
        scratch_types=(
            [pltpu.VMEM((T,), jnp.int32),
             pltpu.VMEM((NCHUNK, CHUNK), jnp.int32)]
            + [pltpu.VMEM((CHUNK, D), jnp.float32)] * DEPTH
            + [pltpu.SemaphoreType.DMA] * (2 * DEPTH)
        ),
    )(table, idx)
    return out.reshape(B, T, D)


# speedup vs baseline: 9.7398x; 9.7398x over previous
import jax
import jax.numpy as jnp
from jax import lax
from jax.experimental import pallas as pl
from jax.experimental.pallas import tpu as pltpu
from jax.experimental.pallas import tpu_sc as plsc

B = 32
S = 256
T = 1024
MASK = 768
D = 768
ROWS = B * S
NMST = 1024
CHUNK = 64
NCHUNK = T // CHUNK
DEPTH = 2
LANES = 16
NC = 2
NS = 16


def _body(table, idx, out, *refs):
    idx_v = refs[0]
    srcs = refs[1:1 + NCHUNK]
    bufs = refs[1 + NCHUNK:1 + NCHUNK + DEPTH]
    gsems = refs[1 + NCHUNK + DEPTH:1 + NCHUNK + 2 * DEPTH]
    wsems = refs[1 + NCHUNK + 2 * DEPTH:]
    wid = lax.axis_index("s") * NC + lax.axis_index("c")
    pltpu.sync_copy(idx, idx_v)

    in_base = wid * S - MASK
    mst_base = (wid * 2 * LANES) % NMST
    for i in range(T // LANES):
        v = idx_v[pl.ds(i * LANES, LANES)]
        j = lax.iota(jnp.int32, LANES) + (i * LANES)
        mst_row = ((j + mst_base) & (NMST - 1)) + ROWS
        src = jnp.where(v < MASK, mst_row, v + in_base)
        srcs[i * LANES // CHUNK][pl.ds((i * LANES) % CHUNK, LANES)] = src

    gathers = [None] * NCHUNK
    writes = [None] * NCHUNK

    def start_gather(c):
        gathers[c] = pltpu.async_copy(
            table.at[srcs[c]], bufs[c % DEPTH], gsems[c % DEPTH])

    for c in range(min(DEPTH, NCHUNK)):
        start_gather(c)
    for c in range(NCHUNK):
        gathers[c].wait()
        writes[c] = pltpu.async_copy(
            bufs[c % DEPTH], out.at[pl.ds(wid * T + c * CHUNK, CHUNK)],
            wsems[c % DEPTH])
        nxt = c + DEPTH
        if nxt < NCHUNK:
            writes[c].wait()
            start_gather(nxt)
        else:
            writes[c].wait()


def kernel(inputs, mask_indices, un_masked_indices, mst):
    idx = jnp.concatenate([mask_indices, un_masked_indices]).astype(jnp.int32)
    table = jnp.concatenate(
        [inputs.reshape(ROWS, D),
         jnp.broadcast_to(mst.reshape(1, D).astype(inputs.dtype), (NMST, D))],
        axis=0)
    mesh = plsc.VectorSubcoreMesh(core_axis_name="c", subcore_axis_name="s")
    out = pl.kernel(
        _body,
        mesh=mesh,
        out_type=jax.ShapeDtypeStruct((B * T, D), inputs.dtype),
        scratch_types=(
            [pltpu.VMEM((T,), jnp.int32)]
            + [pltpu.VMEM((CHUNK,), jnp.int32)] * NCHUNK
            + [pltpu.VMEM((CHUNK, D), jnp.float32)] * DEPTH
            + [pltpu.SemaphoreType.DMA] * (2 * DEPTH)
        ),
    )(table, idx)
    return out.reshape(B, T, D)
